# Initial kernel scaffold; baseline (speedup 1.0000x reference)
import jax
import jax.numpy as jnp
from jax.experimental import pallas as pl


def kernel(x, roi_in, up_rows, up_cols, up_vals, indices, W_ref, b_ref, W_se1, W_se2, W_c1, b_c1, W_d3, b_d3, W_2d3, b_2d3, W_c, b_c):
    raise NotImplementedError("write your pallas kernel here")


# speedup vs baseline: 2.1550x; 2.1550x over previous
import functools

import jax
import jax.numpy as jnp
from jax import lax
from jax.experimental import pallas as pl
from jax.experimental.pallas import tpu as pltpu
from jax.experimental.pallas import tpu_sc as plsc

B, N_IN, N_OUT, C, L = 8, 2500, 10000, 128, 9
POSE = 225
NNZ = 30000
OUT_CH = 128

NC, NS = 2, 16
NW = NC * NS
WPB = NW // B

A_CHUNK = 128
A_CHUNKS_PER_SUB = 16
NNZ_PAD = NS * A_CHUNKS_PER_SUB * A_CHUNK
BATCH_PER_SC = B // NC
W_CHUNK = 200
HALF = N_OUT // 2
POOL_ROWS = HALF + W_CHUNK
ZP_CH = POOL_ROWS // W_CHUNK
FL_CH = HALF // W_CHUNK

G_CHUNK = 8
G_ROWS = G_CHUNK * L
N_CHUNKS = N_OUT // G_CHUNK

@functools.cache
def _mesh():
    return plsc.VectorSubcoreMesh(core_axis_name="c", subcore_axis_name="s",
                                  num_cores=NC, num_subcores=NS)


def _pool_body(x_hbm, cols_hbm, rows_hbm, vals_hbm, pool_hbm,
               colbuf, rowbuf, valbuf, xbuf, zbuf, pool_sp, sem):
    cid = lax.axis_index("c")
    sid = lax.axis_index("s")
    zero = jnp.zeros((16,), jnp.float32)

    @pl.loop(0, W_CHUNK)
    def _zero_zbuf(i):
        for j in range(C // 16):
            zbuf[i, pl.ds(j * 16, 16)] = zero

    lanes = lax.iota(jnp.int32, 16)
    for bi in range(BATCH_PER_SC):
        b = cid * BATCH_PER_SC + bi
        for h in range(2):

            @pl.loop(sid, ZP_CH, step=NS)
            def _zero(t):
                pltpu.sync_copy(zbuf, pool_sp.at[pl.ds(t * W_CHUNK, W_CHUNK)])

            plsc.subcore_barrier()

            @pl.loop(0, A_CHUNKS_PER_SUB)
            def _chunk(k):
                base = sid * (A_CHUNKS_PER_SUB * A_CHUNK) + k * A_CHUNK
                pltpu.sync_copy(cols_hbm.at[pl.ds(base, A_CHUNK)], colbuf)
                pltpu.sync_copy(rows_hbm.at[pl.ds(base, A_CHUNK)], rowbuf)
                pltpu.sync_copy(vals_hbm.at[pl.ds(base, A_CHUNK)], valbuf)
                for j in range(A_CHUNK // 16):
                    sl = pl.ds(j * 16, 16)
                    rv = rowbuf[sl] - (h * HALF)
                    ok = (rv >= 0) & (rv < HALF)
                    rowbuf[sl] = jnp.where(ok, rv, HALF + j * 16 + lanes)
                pltpu.async_copy(x_hbm.at[b].at[colbuf], xbuf, sem).wait()

                @pl.loop(0, A_CHUNK)
                def _scale(i):
                    vv = valbuf[i, pl.ds(0, 16)]
                    for j in range(C // 16):
                        sl = pl.ds(j * 16, 16)
                        xbuf[i, sl] = xbuf[i, sl] * vv

                pltpu.sync_copy(xbuf, pool_sp.at[rowbuf], add=True)

            plsc.subcore_barrier()

            @pl.loop(sid, FL_CH, step=NS)
            def _flush(t):
                pltpu.sync_copy(
                    pool_sp.at[pl.ds(t * W_CHUNK, W_CHUNK)],
                    pool_hbm.at[b, pl.ds(h * HALF + t * W_CHUNK, W_CHUNK)])

            plsc.subcore_barrier()


@functools.cache
def _pool_call():
    return pl.kernel(
        _pool_body,
        out_type=jax.ShapeDtypeStruct((B, N_OUT, C), jnp.float32),
        mesh=_mesh(),
        scratch_types=[
            pltpu.VMEM((A_CHUNK,), jnp.int32),
            pltpu.VMEM((A_CHUNK,), jnp.int32),
            pltpu.VMEM((A_CHUNK, 16), jnp.float32),
            pltpu.VMEM((A_CHUNK, C), jnp.float32),
            pltpu.VMEM((W_CHUNK, C), jnp.float32),
            pltpu.VMEM_SHARED((POOL_ROWS, C), jnp.float32),
            pltpu.SemaphoreType.DMA,
        ],
    )


def _gather_body(relu, partials, z_hbm, idx_hbm, add_hbm, *refs):
    if partials:
        out_hbm, part_hbm, idxbuf, gbuf, obuf, avbuf, accbuf, sem = refs
    else:
        out_hbm, idxbuf, gbuf, obuf, avbuf, accbuf, sem = refs
    cid = lax.axis_index("c")
    sid = lax.axis_index("s")
    w = sid * NC + cid
    b = w // WPB
    q = w % WPB

    pltpu.sync_copy(add_hbm.at[b], avbuf)
    if partials:
        zero = jnp.zeros((16,), jnp.float32)
        for j in range(C // 16):
            accbuf[0, pl.ds(j * 16, 16)] = zero

    @pl.loop(q, N_CHUNKS, step=WPB)
    def _chunk(ch):
        pltpu.sync_copy(idx_hbm.at[pl.ds(ch * G_ROWS, G_ROWS)], idxbuf)
        pltpu.async_copy(z_hbm.at[b].at[idxbuf], gbuf, sem).wait()
        for r in range(G_CHUNK):
            for j in range(C // 16):
                sl = pl.ds(j * 16, 16)
                acc = gbuf[r * L, sl]
                for l in range(1, L):
                    acc = acc + gbuf[r * L + l, sl]
                acc = acc + avbuf[0, sl]
                if relu:
                    acc = jnp.maximum(acc, 0.0)
                obuf[r, sl] = acc
                if partials:
                    accbuf[0, sl] = accbuf[0, sl] + acc
        pltpu.sync_copy(obuf, out_hbm.at[b, pl.ds(ch * G_CHUNK, G_CHUNK)])

    if partials:
        pltpu.sync_copy(accbuf, part_hbm.at[b, q])


@functools.cache
def _make_gather_call(relu, partials):
    out_type = [jax.ShapeDtypeStruct((B, N_OUT, C), jnp.float32)]
    if partials:
        out_type.append(jax.ShapeDtypeStruct((B, WPB, 1, C), jnp.float32))
    return pl.kernel(
        functools.partial(_gather_body, relu, partials),
        out_type=out_type,
        mesh=_mesh(),
        scratch_types=[
            pltpu.VMEM((G_ROWS,), jnp.int32),
            pltpu.VMEM((G_ROWS, C), jnp.float32),
            pltpu.VMEM((G_CHUNK, C), jnp.float32),
            pltpu.VMEM((1, C), jnp.float32),
            pltpu.VMEM((1, C), jnp.float32),
            pltpu.SemaphoreType.DMA,
        ],
    )


BN = 400


def _z_tc_body(pool_ref, w_ref, roi_ref, wps_ref, bref_ref, z_ref, pose_ref):
    z_ref[...] = jnp.dot(pool_ref[0], w_ref[...],
                         preferred_element_type=jnp.float32)[None]

    @pl.when(pl.program_id(1) == 0)
    def _():
        pose_ref[...] = (
            jnp.dot(roi_ref[0], wps_ref[...],
                    preferred_element_type=jnp.float32) + bref_ref[...])[None]


def _z_tc_call(pool, w_pool_cat, roi_flat, w_pose_sum, b_ref_row):
    grid = (B, N_OUT // BN)
    return pl.pallas_call(
        _z_tc_body,
        grid=grid,
        in_specs=[
            pl.BlockSpec((1, BN, C), lambda b, n: (b, n, 0)),
            pl.BlockSpec((C, L * C), lambda b, n: (0, 0)),
            pl.BlockSpec((1, 1, POSE), lambda b, n: (b, 0, 0)),
            pl.BlockSpec((POSE, C), lambda b, n: (0, 0)),
            pl.BlockSpec((1, C), lambda b, n: (0, 0)),
        ],
        out_specs=[
            pl.BlockSpec((1, BN, L * C), lambda b, n: (b, n, 0)),
            pl.BlockSpec((1, 1, C), lambda b, n: (b, 0, 0)),
        ],
        out_shape=[
            jax.ShapeDtypeStruct((B, N_OUT, L * C), jnp.float32),
            jax.ShapeDtypeStruct((B, 1, C), jnp.float32),
        ],
    )(pool, w_pool_cat, roi_flat[:, None, :], w_pose_sum, b_ref_row)


def _y_tc_body(mid_ref, part_ref, wse1_ref, wse2_ref, wbig_ref, y_ref):
    s = jnp.sum(part_ref[0], axis=(0, 1), keepdims=False)[None] * (1.0 / N_OUT)
    h = jnp.maximum(
        jnp.dot(s, wse1_ref[...], preferred_element_type=jnp.float32), 0.0)
    yv = jax.nn.sigmoid(
        jnp.dot(h, wse2_ref[...], preferred_element_type=jnp.float32))
    scaled = mid_ref[0] * yv
    y_ref[...] = jnp.dot(scaled, wbig_ref[...],
                         preferred_element_type=jnp.float32)[None]


def _y_tc_call(mid, parts, w_se1, w_se2, w_big_cat):
    grid = (B, N_OUT // BN)
    return pl.pallas_call(
        _y_tc_body,
        grid=grid,
        in_specs=[
            pl.BlockSpec((1, BN, C), lambda b, n: (b, n, 0)),
            pl.BlockSpec((1, WPB, 1, C), lambda b, n: (b, 0, 0, 0)),
            pl.BlockSpec((C, C // 16), lambda b, n: (0, 0)),
            pl.BlockSpec((C // 16, C), lambda b, n: (0, 0)),
            pl.BlockSpec((C, L * OUT_CH), lambda b, n: (0, 0)),
        ],
        out_specs=pl.BlockSpec((1, BN, L * OUT_CH), lambda b, n: (b, n, 0)),
        out_shape=jax.ShapeDtypeStruct((B, N_OUT, L * OUT_CH), jnp.float32),
    )(mid, parts, w_se1, w_se2, w_big_cat)


def kernel(x, roi_in, up_rows, up_cols, up_vals, indices, W_ref, b_ref,
           W_se1, W_se2, W_c1, b_c1, W_d3, b_d3, W_2d3, b_2d3, W_c, b_c):
    CP = C + POSE
    Wr3 = W_ref.reshape(L, CP, C)
    w_pool_cat = Wr3[:, :C, :].transpose(1, 0, 2).reshape(C, L * C)
    w_pose_sum = Wr3[:, C:, :].sum(axis=0)

    wc3 = W_c.reshape(L, C, OUT_CH // 2)
    w2d3 = jnp.pad(W_2d3.reshape(2 * L // 3, C, OUT_CH // 4),
                   ((0, L // 3), (0, 0), (0, 0)))
    wd3 = jnp.pad(W_d3.reshape(L // 3, C, OUT_CH // 4),
                  ((0, 2 * L // 3), (0, 0), (0, 0)))
    w_big = jnp.concatenate([wc3, w2d3, wd3], axis=2)
    w_big = w_big.at[0].add(W_c1)
    w_big_cat = w_big.transpose(1, 0, 2).reshape(C, L * OUT_CH)
    b_tot = b_c1 + jnp.concatenate([b_c, b_2d3, b_d3])

    idx9 = (indices * L + jnp.arange(L, dtype=jnp.int32)[None, :]).reshape(-1)
    cols_p = jnp.pad(up_cols, (0, NNZ_PAD - NNZ))
    rows_p = jnp.pad(up_rows, (0, NNZ_PAD - NNZ))
    vals_p = jnp.broadcast_to(
        jnp.pad(up_vals, (0, NNZ_PAD - NNZ))[:, None], (NNZ_PAD, 16))
    roi_flat = roi_in.reshape(B, POSE)

    pool = _pool_call()(x, cols_p, rows_p, vals_p)
    z, pose = _z_tc_call(pool, w_pool_cat, roi_flat, w_pose_sum,
                         b_ref.reshape(1, C))
    zf = z.reshape(B, N_OUT * L, C)
    mid, parts = _make_gather_call(False, True)(zf, idx9, pose)
    y = _y_tc_call(mid, parts, W_se1, W_se2, w_big_cat)
    yf = y.reshape(B, N_OUT * L, C)
    (out,) = _make_gather_call(True, False)(
        yf, idx9, jnp.tile(b_tot[None, None], (B, 1, 1)))
    return out
